# baseline (speedup 1.0000x reference)
import jax
import jax.numpy as jnp
from jax import lax
from jax.experimental import pallas as pl
from jax.experimental.pallas import tpu as pltpu
from jax.experimental.pallas import tpu_sc as plsc

RS = 3.0
RC = 6.0
N_NODES = 100000
N_EDGES = 1600000
D_EMB = 5
D = D_EMB + 3

NC = 2
NS = 16
NW = NC * NS
L = 16

NPW = N_NODES // NW
ACC_C = 22
ACC_PAD = NPW * ACC_C + 2
EB = 2048
SUB = EB // L
NCHUNK = 625
NGROUP = (NCHUNK + L - 1) // L

GB = 128
EPW_A = 50048
NB_A = EPW_A // GB
NB_LAST = (N_EDGES - (NW - 1) * EPW_A) // GB

_MESH = plsc.VectorSubcoreMesh(core_axis_name="c", subcore_axis_name="s",
                               num_cores=NC, num_subcores=NS)
_PARAMS = pltpu.CompilerParams(use_tc_tiling_on_sc=False,
                               needs_layout_passes=False)


def _rsqrt(t):
    i = plsc.bitcast(t, jnp.int32)
    i = 0x5F3759DF - lax.shift_right_arithmetic(i, 1)
    y = plsc.bitcast(i, jnp.float32)
    for _ in range(4):
        y = y * (1.5 - 0.5 * t * y * y)
    return y


def _aggregate_body(vx_hbm, vy_hbm, vz_hbm, eidx_hbm, wb_hbm, bounds_hbm,
                    node_hbm, vx_v, vy_v, vz_v, eidx_v, wb_v, bounds_v, acc_v,
                    stage_v, semld):
    wid = lax.axis_index("s") * NC + lax.axis_index("c")
    n0 = wid * NPW
    iota = lax.iota(jnp.int32, L)

    pltpu.sync_copy(wb_hbm, wb_v)
    pltpu.sync_copy(bounds_hbm, bounds_v)

    b0 = bounds_v[pl.ds(0, L)]
    b1 = bounds_v[pl.ds(L, L)]
    b2 = bounds_v[pl.ds(2 * L, L)]

    def extract(j):
        vec = jnp.where(j < L, b0, jnp.where(j < 2 * L, b1, b2))
        return jnp.max(jnp.where(iota == j % L, vec, 0))

    e_lo = extract(wid)
    e_hi = extract(wid + 1)
    e_start = (e_lo // 8) * 8
    nb = (e_hi - e_start + EB - 1) // EB

    def zero(i, _):
        acc_v[pl.ds(i * L, L)] = jnp.zeros((L,), jnp.float32)
        return ()
    lax.fori_loop(0, ACC_PAD // L, zero, (), unroll=4)

    wrow = [wb_v[d] for d in range(10)]

    def start_load(i, slot):
        e0 = e_start + i * EB
        for hbm, v in ((vx_hbm, vx_v), (vy_hbm, vy_v), (vz_hbm, vz_v),
                       (eidx_hbm, eidx_v)):
            pltpu.async_copy(hbm.at[pl.ds(e0, EB)], v.at[slot], semld.at[slot])

    def drain_load(i, slot):
        e0 = e_start + i * EB
        for hbm, v in ((vx_hbm, vx_v), (vy_hbm, vy_v), (vz_hbm, vz_v),
                       (eidx_hbm, eidx_v)):
            pltpu.make_async_copy(hbm.at[pl.ds(e0, EB)], v.at[slot],
                                  semld.at[slot]).wait()

    @pl.when(nb > 0)
    def _():
        start_load(0, 0)

    lane0 = iota * SUB

    def chunk(i, _):
        slot = lax.rem(i, 2)
        nslot = lax.rem(i + 1, 2)

        @pl.when(i + 1 < nb)
        def _():
            start_load(i + 1, nslot)

        drain_load(i, slot)
        e0 = e_start + i * EB
        slot_vec = jnp.zeros((L,), jnp.int32) + slot

        def group(j, _):
            row = lane0 + j
            eidx = plsc.load_gather(eidx_v, [slot_vec, row])
            eg = e0 + row
            m = jnp.logical_and(eg >= e_lo, eg < e_hi)
            m_f = jnp.where(m, 1.0, 0.0)
            lidx = jnp.clip(eidx - n0, 0, NPW - 1)

            vx = plsc.load_gather(vx_v, [slot_vec, row])
            vy = plsc.load_gather(vy_v, [slot_vec, row])
            vz = plsc.load_gather(vz_v, [slot_vec, row])

            t = vx * vx + vy * vy + vz * vz
            inv_r = _rsqrt(t)
            r = t * inv_r
            xq = (r - RC) * (1.0 / (RS - RC))
            poly = xq * xq * xq * (10.0 + xq * (-15.0 + 6.0 * xq)) + 1.0
            s = jnp.where(r < RS, inv_r,
                          jnp.where(r < RC, inv_r * poly,
                                    jnp.zeros((L,), jnp.float32)))

            vxm = vx * m_f
            vym = vy * m_f
            vzm = vz * m_f
            base = lidx * ACC_C
            vals = []
            for d in range(D_EMB):
                emb = s * wrow[d] + wrow[D_EMB + d]
                vals += [emb * vxm, emb * vym, emb * vzm]
            vals += [vxm * vx, vxm * vy, vxm * vz, vym * vy, vym * vz,
                     vzm * vz, m_f]
            for j2, v in enumerate(vals):
                plsc.addupdate_scatter(acc_v, [base + j2], v)
            return ()

        lax.fori_loop(0, SUB, group, ())
        return ()

    lax.fori_loop(0, nb, chunk, ())

    def out_chunk(c, _):
        def col_of(d, e):
            return jnp.full((L,), d * D + e, jnp.int32)

        def group(g, _):
            nl = c * NCHUNK + g * L + iota
            lane_ok = (g * L + iota) < NCHUNK
            nl_c = jnp.clip(nl, 0, NPW - 1)
            base = nl_c * ACC_C
            sums = [plsc.load_gather(acc_v, [base + j]) for j in range(ACC_C)]
            cnt = sums[21]
            inv = 1.0 / jnp.maximum(cnt, 1.0)
            mm = [sj * inv for sj in sums[:21]]
            ax = [mm[3 * d] for d in range(D_EMB)] + [mm[15], mm[16], mm[17]]
            ay = [mm[3 * d + 1] for d in range(D_EMB)] + [mm[16], mm[18], mm[19]]
            az = [mm[3 * d + 2] for d in range(D_EMB)] + [mm[17], mm[19], mm[20]]
            srow = g * L + iota
            for d in range(D):
                for e in range(d, D):
                    val = ax[d] * ax[e] + ay[d] * ay[e] + az[d] * az[e]
                    plsc.store_scatter(stage_v, [srow, col_of(d, e)], val,
                                       mask=lane_ok)
                    if e != d:
                        plsc.store_scatter(stage_v, [srow, col_of(e, d)], val,
                                           mask=lane_ok)
            return ()

        lax.fori_loop(0, NGROUP, group, ())
        pltpu.sync_copy(stage_v,
                        node_hbm.at[pl.ds(n0 + c * NCHUNK, NCHUNK)])
        return ()

    lax.fori_loop(0, NPW // NCHUNK, out_chunk, ())


def _aggregate(vx_pad, vy_pad, vz_pad, eidx_pad, wb2d, bounds):
    fn = pl.kernel(
        _aggregate_body,
        out_type=jax.ShapeDtypeStruct((N_NODES, D * D), jnp.float32),
        mesh=_MESH,
        compiler_params=_PARAMS,
        scratch_types=[
            pltpu.VMEM((2, EB), jnp.float32),
            pltpu.VMEM((2, EB), jnp.float32),
            pltpu.VMEM((2, EB), jnp.float32),
            pltpu.VMEM((2, EB), jnp.int32),
            pltpu.VMEM((L, L), jnp.float32),
            pltpu.VMEM((3 * L,), jnp.int32),
            pltpu.VMEM((ACC_PAD,), jnp.float32),
            pltpu.VMEM((NCHUNK, D * D), jnp.float32),
            pltpu.SemaphoreType.DMA((2,)),
        ],
    )
    return fn(vx_pad, vy_pad, vz_pad, eidx_pad, wb2d, bounds)


def _edge_update_body(node_hbm, ei0_hbm, ei1_hbm, out_hbm, idx0_v, idx1_v,
                      rows0_v, rows1_v, ostage_v, semi, semg, semo):
    wid = lax.axis_index("s") * NC + lax.axis_index("c")
    base = wid * EPW_A
    nb = jnp.where(wid < NW - 1, NB_A, NB_LAST)
    iota = lax.iota(jnp.int32, L)

    def start_idx(i, slot):
        off = base + i * GB
        pltpu.async_copy(ei0_hbm.at[pl.ds(off, GB)], idx0_v.at[slot],
                         semi.at[slot])
        pltpu.async_copy(ei1_hbm.at[pl.ds(off, GB)], idx1_v.at[slot],
                         semi.at[slot])

    def drain_idx(i, slot):
        off = base + i * GB
        pltpu.make_async_copy(ei0_hbm.at[pl.ds(off, GB)], idx0_v.at[slot],
                              semi.at[slot]).wait()
        pltpu.make_async_copy(ei1_hbm.at[pl.ds(off, GB)], idx1_v.at[slot],
                              semi.at[slot]).wait()

    def start_gathers(slot):
        pltpu.async_copy(node_hbm.at[idx0_v.at[slot]], rows0_v.at[slot],
                         semg.at[slot])
        pltpu.async_copy(node_hbm.at[idx1_v.at[slot]], rows1_v.at[slot],
                         semg.at[slot])

    def drain_gathers(slot):
        pltpu.make_async_copy(node_hbm.at[idx0_v.at[slot]], rows0_v.at[slot],
                              semg.at[slot]).wait()
        pltpu.make_async_copy(node_hbm.at[idx1_v.at[slot]], rows1_v.at[slot],
                              semg.at[slot]).wait()

    start_idx(0, 0)
    drain_idx(0, 0)
    start_gathers(0)

    @pl.when(nb > 1)
    def _():
        start_idx(1, 1)

    def batch(i, _):
        slot = lax.rem(i, 2)
        nslot = lax.rem(i + 1, 2)

        drain_gathers(slot)

        @pl.when(i + 2 < nb)
        def _():
            start_idx(i + 2, slot)

        @pl.when(i + 1 < nb)
        def _():
            drain_idx(i + 1, nslot)
            start_gathers(nslot)

        @pl.when(i >= 2)
        def _():
            off_prev = base + (i - 2) * GB
            pltpu.make_async_copy(ostage_v.at[slot],
                                  out_hbm.at[:, pl.ds(off_prev, GB)],
                                  semo.at[slot]).wait()

        slot_vec = jnp.zeros((L,), jnp.int32) + slot

        def add_row(r, _):
            rvec = jnp.zeros((L,), jnp.int32) + r
            for k in range(4):
                sl = pl.ds(k * L, L)
                val = rows0_v[slot, r, sl] + rows1_v[slot, r, sl]
                plsc.store_scatter(ostage_v, [slot_vec, k * L + iota, rvec],
                                   val)
            return ()

        lax.fori_loop(0, GB, add_row, (), unroll=2)
        off = base + i * GB
        pltpu.async_copy(ostage_v.at[slot], out_hbm.at[:, pl.ds(off, GB)],
                         semo.at[slot])
        return ()

    lax.fori_loop(0, nb, batch, ())

    def final_drain(k):
        i_last = nb - 2 + k
        real_slot = lax.rem(i_last, 2)

        @pl.when(i_last >= 0)
        def _():
            off = base + i_last * GB
            pltpu.make_async_copy(ostage_v.at[real_slot],
                                  out_hbm.at[:, pl.ds(off, GB)],
                                  semo.at[real_slot]).wait()

    final_drain(0)
    final_drain(1)


def _edge_update(node128, ei0, ei1):
    fn = pl.kernel(
        _edge_update_body,
        out_type=jax.ShapeDtypeStruct((D * D, N_EDGES), jnp.float32),
        mesh=_MESH,
        compiler_params=pltpu.CompilerParams(needs_layout_passes=False),
        scratch_types=[
            pltpu.VMEM((2, GB), jnp.int32),
            pltpu.VMEM((2, GB), jnp.int32),
            pltpu.VMEM((2, GB, 2 * D * D), jnp.float32),
            pltpu.VMEM((2, GB, 2 * D * D), jnp.float32),
            pltpu.VMEM((2, D * D, GB), jnp.float32),
            pltpu.SemaphoreType.DMA((2,)),
            pltpu.SemaphoreType.DMA((2,)),
            pltpu.SemaphoreType.DMA((2,)),
        ],
    )
    return fn(node128, ei0, ei1)


def kernel(env_vectors, env_index, edge_index, W_emb, b_emb):
    vx_pad = jnp.pad(env_vectors[:, 0], (0, EB))
    vy_pad = jnp.pad(env_vectors[:, 1], (0, EB))
    vz_pad = jnp.pad(env_vectors[:, 2], (0, EB))
    eidx_pad = jnp.pad(env_index, (0, EB), constant_values=N_NODES)
    wb = jnp.concatenate([W_emb.reshape(-1), b_emb.reshape(-1)])
    wb2d = jnp.tile(wb[:, None], (1, L))
    wb2d = jnp.pad(wb2d, ((0, L - 10), (0, 0)))
    bounds = jnp.searchsorted(env_index,
                              jnp.arange(NW + 1, dtype=jnp.int32) * NPW
                              ).astype(jnp.int32)
    bounds = jnp.pad(bounds, (0, 3 * L - (NW + 1)), constant_values=N_EDGES)

    node = _aggregate(vx_pad, vy_pad, vz_pad, eidx_pad, wb2d, bounds)
    node128 = jnp.pad(node, ((0, 0), (0, D * D)))
    edge_t = _edge_update(node128, edge_index[0], edge_index[1])
    return node, edge_t.T


# speedup vs baseline: 1.3127x; 1.3127x over previous
import jax
import jax.numpy as jnp
from jax import lax
from jax.experimental import pallas as pl
from jax.experimental.pallas import tpu as pltpu
from jax.experimental.pallas import tpu_sc as plsc

RS = 3.0
RC = 6.0
N_NODES = 100000
N_EDGES = 1600000
D_EMB = 5
D = D_EMB + 3

NC = 2
NS = 16
NW = NC * NS
L = 16

NPW = N_NODES // NW
ACC_C = 22
ACC_PAD = NPW * ACC_C + 2
EB = 2048
SUB = EB // L
NCHUNK = 625
NGROUP = (NCHUNK + L - 1) // L

GB = 128
EPW_A = 50048
NB_A = EPW_A // GB
NB_LAST = (N_EDGES - (NW - 1) * EPW_A) // GB

_MESH = plsc.VectorSubcoreMesh(core_axis_name="c", subcore_axis_name="s",
                               num_cores=NC, num_subcores=NS)
_PARAMS = pltpu.CompilerParams(use_tc_tiling_on_sc=False,
                               needs_layout_passes=False)


def _rsqrt(t):
    i = plsc.bitcast(t, jnp.int32)
    i = 0x5F3759DF - lax.shift_right_arithmetic(i, 1)
    y = plsc.bitcast(i, jnp.float32)
    for _ in range(4):
        y = y * (1.5 - 0.5 * t * y * y)
    return y


def _aggregate_body(vx_hbm, vy_hbm, vz_hbm, eidx_hbm, wb_hbm, bounds_hbm,
                    node_hbm, vx_v, vy_v, vz_v, eidx_v, wb_v, bounds_v, acc_v,
                    stage_v, semld):
    wid = lax.axis_index("s") * NC + lax.axis_index("c")
    n0 = wid * NPW
    iota = lax.iota(jnp.int32, L)

    pltpu.sync_copy(wb_hbm, wb_v)
    pltpu.sync_copy(bounds_hbm, bounds_v)

    b0 = bounds_v[pl.ds(0, L)]
    b1 = bounds_v[pl.ds(L, L)]
    b2 = bounds_v[pl.ds(2 * L, L)]

    def extract(j):
        vec = jnp.where(j < L, b0, jnp.where(j < 2 * L, b1, b2))
        return jnp.max(jnp.where(iota == j % L, vec, 0))

    e_lo = extract(wid)
    e_hi = extract(wid + 1)
    e_start = (e_lo // 8) * 8
    nb = (e_hi - e_start + EB - 1) // EB

    def zero(i, _):
        acc_v[pl.ds(i * L, L)] = jnp.zeros((L,), jnp.float32)
        return ()
    lax.fori_loop(0, ACC_PAD // L, zero, (), unroll=4)

    wrow = [wb_v[d] for d in range(10)]

    def start_load(i, slot):
        e0 = e_start + i * EB
        for hbm, v in ((vx_hbm, vx_v), (vy_hbm, vy_v), (vz_hbm, vz_v),
                       (eidx_hbm, eidx_v)):
            pltpu.async_copy(hbm.at[pl.ds(e0, EB)], v.at[slot], semld.at[slot])

    def drain_load(i, slot):
        e0 = e_start + i * EB
        for hbm, v in ((vx_hbm, vx_v), (vy_hbm, vy_v), (vz_hbm, vz_v),
                       (eidx_hbm, eidx_v)):
            pltpu.make_async_copy(hbm.at[pl.ds(e0, EB)], v.at[slot],
                                  semld.at[slot]).wait()

    @pl.when(nb > 0)
    def _():
        start_load(0, 0)

    lane0 = iota * SUB

    def chunk(i, _):
        slot = lax.rem(i, 2)
        nslot = lax.rem(i + 1, 2)

        @pl.when(i + 1 < nb)
        def _():
            start_load(i + 1, nslot)

        drain_load(i, slot)
        e0 = e_start + i * EB
        slot_vec = jnp.zeros((L,), jnp.int32) + slot

        def group(j, _):
            row = lane0 + j
            eidx = plsc.load_gather(eidx_v, [slot_vec, row])
            eg = e0 + row
            m = jnp.logical_and(eg >= e_lo, eg < e_hi)
            m_f = jnp.where(m, 1.0, 0.0)
            lidx = jnp.clip(eidx - n0, 0, NPW - 1)

            vx = plsc.load_gather(vx_v, [slot_vec, row])
            vy = plsc.load_gather(vy_v, [slot_vec, row])
            vz = plsc.load_gather(vz_v, [slot_vec, row])

            t = vx * vx + vy * vy + vz * vz
            inv_r = _rsqrt(t)
            r = t * inv_r
            xq = (r - RC) * (1.0 / (RS - RC))
            poly = xq * xq * xq * (10.0 + xq * (-15.0 + 6.0 * xq)) + 1.0
            s = jnp.where(r < RS, inv_r,
                          jnp.where(r < RC, inv_r * poly,
                                    jnp.zeros((L,), jnp.float32)))

            vxm = vx * m_f
            vym = vy * m_f
            vzm = vz * m_f
            base = lidx * ACC_C
            vals = []
            for d in range(D_EMB):
                emb = s * wrow[d] + wrow[D_EMB + d]
                vals += [emb * vxm, emb * vym, emb * vzm]
            vals += [vxm * vx, vxm * vy, vxm * vz, vym * vy, vym * vz,
                     vzm * vz, m_f]
            for j2, v in enumerate(vals):
                plsc.addupdate_scatter(acc_v, [base + j2], v)
            return ()

        lax.fori_loop(0, SUB, group, ())
        return ()

    lax.fori_loop(0, nb, chunk, ())

    def out_chunk(c, _):
        def col_of(d, e):
            return jnp.full((L,), d * D + e, jnp.int32)

        def group(g, _):
            nl = c * NCHUNK + g * L + iota
            lane_ok = (g * L + iota) < NCHUNK
            nl_c = jnp.clip(nl, 0, NPW - 1)
            base = nl_c * ACC_C
            sums = [plsc.load_gather(acc_v, [base + j]) for j in range(ACC_C)]
            cnt = sums[21]
            inv = 1.0 / jnp.maximum(cnt, 1.0)
            mm = [sj * inv for sj in sums[:21]]
            ax = [mm[3 * d] for d in range(D_EMB)] + [mm[15], mm[16], mm[17]]
            ay = [mm[3 * d + 1] for d in range(D_EMB)] + [mm[16], mm[18], mm[19]]
            az = [mm[3 * d + 2] for d in range(D_EMB)] + [mm[17], mm[19], mm[20]]
            srow = g * L + iota
            for d in range(D):
                for e in range(d, D):
                    val = ax[d] * ax[e] + ay[d] * ay[e] + az[d] * az[e]
                    plsc.store_scatter(stage_v, [srow, col_of(d, e)], val,
                                       mask=lane_ok)
                    if e != d:
                        plsc.store_scatter(stage_v, [srow, col_of(e, d)], val,
                                           mask=lane_ok)
            return ()

        lax.fori_loop(0, NGROUP, group, ())
        pltpu.sync_copy(stage_v,
                        node_hbm.at[pl.ds(n0 + c * NCHUNK, NCHUNK)])
        return ()

    lax.fori_loop(0, NPW // NCHUNK, out_chunk, ())


def _aggregate(vx_pad, vy_pad, vz_pad, eidx_pad, wb2d, bounds):
    fn = pl.kernel(
        _aggregate_body,
        out_type=jax.ShapeDtypeStruct((N_NODES, D * D), jnp.float32),
        mesh=_MESH,
        compiler_params=_PARAMS,
        scratch_types=[
            pltpu.VMEM((2, EB), jnp.float32),
            pltpu.VMEM((2, EB), jnp.float32),
            pltpu.VMEM((2, EB), jnp.float32),
            pltpu.VMEM((2, EB), jnp.int32),
            pltpu.VMEM((L, L), jnp.float32),
            pltpu.VMEM((3 * L,), jnp.int32),
            pltpu.VMEM((ACC_PAD,), jnp.float32),
            pltpu.VMEM((NCHUNK, D * D), jnp.float32),
            pltpu.SemaphoreType.DMA((2,)),
        ],
    )
    return fn(vx_pad, vy_pad, vz_pad, eidx_pad, wb2d, bounds)


def _edge_update_body(node_hbm, ei0_hbm, ei1_hbm, out_hbm, idx0_v, idx1_v,
                      rows0_v, rows1_v, ostage_v, semi, semg, semo):
    wid = lax.axis_index("s") * NC + lax.axis_index("c")
    base = wid * EPW_A
    nb = jnp.where(wid < NW - 1, NB_A, NB_LAST)
    iota = lax.iota(jnp.int32, L)

    def start_idx(i, slot):
        off = base + i * GB
        pltpu.async_copy(ei0_hbm.at[pl.ds(off, GB)], idx0_v.at[slot],
                         semi.at[slot])
        pltpu.async_copy(ei1_hbm.at[pl.ds(off, GB)], idx1_v.at[slot],
                         semi.at[slot])

    def drain_idx(i, slot):
        off = base + i * GB
        pltpu.make_async_copy(ei0_hbm.at[pl.ds(off, GB)], idx0_v.at[slot],
                              semi.at[slot]).wait()
        pltpu.make_async_copy(ei1_hbm.at[pl.ds(off, GB)], idx1_v.at[slot],
                              semi.at[slot]).wait()

    def start_gathers(slot):
        pltpu.async_copy(node_hbm.at[idx0_v.at[slot]], rows0_v.at[slot],
                         semg.at[slot])
        pltpu.async_copy(node_hbm.at[idx1_v.at[slot]], rows1_v.at[slot],
                         semg.at[slot])

    def drain_gathers(slot):
        pltpu.make_async_copy(node_hbm.at[idx0_v.at[slot]], rows0_v.at[slot],
                              semg.at[slot]).wait()
        pltpu.make_async_copy(node_hbm.at[idx1_v.at[slot]], rows1_v.at[slot],
                              semg.at[slot]).wait()

    start_idx(0, 0)
    drain_idx(0, 0)
    start_gathers(0)

    @pl.when(nb > 1)
    def _():
        start_idx(1, 1)

    def batch(i, _):
        slot = lax.rem(i, 2)
        nslot = lax.rem(i + 1, 2)

        drain_gathers(slot)

        @pl.when(i + 2 < nb)
        def _():
            start_idx(i + 2, slot)

        @pl.when(i + 1 < nb)
        def _():
            drain_idx(i + 1, nslot)
            start_gathers(nslot)

        @pl.when(i >= 2)
        def _():
            off_prev = base + (i - 2) * GB
            pltpu.make_async_copy(ostage_v.at[slot],
                                  out_hbm.at[pl.ds(off_prev, GB)],
                                  semo.at[slot]).wait()

        def add_row(r, _):
            for k in range(4):
                sl = pl.ds(k * L, L)
                ostage_v[slot, r, sl] = (rows0_v[slot, r, sl]
                                         + rows1_v[slot, r, sl])
            return ()

        lax.fori_loop(0, GB, add_row, (), unroll=2)
        off = base + i * GB
        pltpu.async_copy(ostage_v.at[slot], out_hbm.at[pl.ds(off, GB)],
                         semo.at[slot])
        return ()

    lax.fori_loop(0, nb, batch, ())

    def final_drain(k):
        i_last = nb - 2 + k
        real_slot = lax.rem(i_last, 2)

        @pl.when(i_last >= 0)
        def _():
            off = base + i_last * GB
            pltpu.make_async_copy(ostage_v.at[real_slot],
                                  out_hbm.at[pl.ds(off, GB)],
                                  semo.at[real_slot]).wait()

    final_drain(0)
    final_drain(1)


def _edge_update(node128, ei0, ei1):
    fn = pl.kernel(
        _edge_update_body,
        out_type=jax.ShapeDtypeStruct((N_EDGES, D * D), jnp.float32),
        mesh=_MESH,
        compiler_params=pltpu.CompilerParams(needs_layout_passes=False),
        scratch_types=[
            pltpu.VMEM((2, GB), jnp.int32),
            pltpu.VMEM((2, GB), jnp.int32),
            pltpu.VMEM((2, GB, 2 * D * D), jnp.float32),
            pltpu.VMEM((2, GB, 2 * D * D), jnp.float32),
            pltpu.VMEM((2, GB, D * D), jnp.float32),
            pltpu.SemaphoreType.DMA((2,)),
            pltpu.SemaphoreType.DMA((2,)),
            pltpu.SemaphoreType.DMA((2,)),
        ],
    )
    return fn(node128, ei0, ei1)


def kernel(env_vectors, env_index, edge_index, W_emb, b_emb):
    vx_pad = jnp.pad(env_vectors[:, 0], (0, EB))
    vy_pad = jnp.pad(env_vectors[:, 1], (0, EB))
    vz_pad = jnp.pad(env_vectors[:, 2], (0, EB))
    eidx_pad = jnp.pad(env_index, (0, EB), constant_values=N_NODES)
    wb = jnp.concatenate([W_emb.reshape(-1), b_emb.reshape(-1)])
    wb2d = jnp.tile(wb[:, None], (1, L))
    wb2d = jnp.pad(wb2d, ((0, L - 10), (0, 0)))
    bounds = jnp.searchsorted(env_index,
                              jnp.arange(NW + 1, dtype=jnp.int32) * NPW
                              ).astype(jnp.int32)
    bounds = jnp.pad(bounds, (0, 3 * L - (NW + 1)), constant_values=N_EDGES)

    node = _aggregate(vx_pad, vy_pad, vz_pad, eidx_pad, wb2d, bounds)
    node128 = jnp.pad(node, ((0, 0), (0, D * D)))
    edge = _edge_update(node128, edge_index[0], edge_index[1])
    return node, edge
